# baseline (speedup 1.0000x reference)
import functools

import jax
import jax.numpy as jnp
from jax import lax
from jax.experimental import pallas as pl
from jax.experimental.pallas import tpu as pltpu


def _round_up(x, m):
    return (x + m - 1) // m * m


def _maybe_pad2d(a, rows, cols):
    r, c = a.shape
    if r == rows and c == cols:
        return a
    return jnp.pad(a, ((0, rows - r), (0, cols - c)))


_DN = (((1,), (1,)), ((), ()))


def _noisy_kernel(x_ref, wmu_ref, wsig_ref, eout_ref, ein_ref,
                  bmu_ref, bsig_ref, beps_ref, o_ref):
    k = pl.program_id(1)
    eo = eout_ref[0, :][:, None]
    eps = eo * ein_ref[...]
    w = (wmu_ref[...] + wsig_ref[...] * eps).astype(jnp.bfloat16)
    xb = x_ref[...].astype(jnp.bfloat16)
    acc = lax.dot_general(xb, w, _DN, preferred_element_type=jnp.float32)

    @pl.when(k == 0)
    def _():
        o_ref[...] = acc + (bmu_ref[...] + bsig_ref[...] * beps_ref[...])

    @pl.when(k != 0)
    def _():
        o_ref[...] += acc


@jax.jit
def kernel(x, weight_mu, weight_sigma, eps_in, eps_out,
           bias_mu, bias_sigma, bias_epsilon):
    x = jnp.asarray(x, jnp.float32)
    B, I = x.shape
    O = bias_mu.shape[0]

    tn = min(_round_up(O, 256), 512)
    M, N = _round_up(B, 8), _round_up(O, tn)
    tk = min(_round_up(I, 128), 1024)
    K = _round_up(I, tk)

    x_p = _maybe_pad2d(x, M, K)
    wmu = _maybe_pad2d(weight_mu, N, K)
    wsig = _maybe_pad2d(weight_sigma, N, K)
    eout = _maybe_pad2d(eps_out.reshape(1, O), 1, N)
    ein = _maybe_pad2d(eps_in.reshape(1, I), 1, K)
    bmu = _maybe_pad2d(bias_mu.reshape(1, O), 1, N)
    bsig = _maybe_pad2d(bias_sigma.reshape(1, O), 1, N)
    beps = _maybe_pad2d(bias_epsilon.reshape(1, O), 1, N)

    row_n = pl.BlockSpec((1, tn), lambda j, k: (0, j))
    grid = (N // tn, K // tk)
    out = pl.pallas_call(
        _noisy_kernel,
        out_shape=jax.ShapeDtypeStruct((M, N), jnp.float32),
        grid=grid,
        in_specs=[
            pl.BlockSpec((M, tk), lambda j, k: (0, k)),
            pl.BlockSpec((tn, tk), lambda j, k: (j, k)),
            pl.BlockSpec((tn, tk), lambda j, k: (j, k)),
            row_n,
            pl.BlockSpec((1, tk), lambda j, k: (0, k)),
            row_n, row_n, row_n,
        ],
        out_specs=pl.BlockSpec((M, tn), lambda j, k: (0, j)),
        compiler_params=pltpu.CompilerParams(
            dimension_semantics=("parallel", "arbitrary")),
    )(x_p, wmu, wsig, eout, ein, bmu, bsig, beps)

    return out[:B, :O]


# speedup vs baseline: 1.2744x; 1.2744x over previous
import functools

import jax
import jax.numpy as jnp
from jax import lax
from jax.experimental import pallas as pl
from jax.experimental.pallas import tpu as pltpu


def _round_up(x, m):
    return (x + m - 1) // m * m


def _maybe_pad2d(a, rows, cols):
    r, c = a.shape
    if r == rows and c == cols:
        return a
    return jnp.pad(a, ((0, rows - r), (0, cols - c)))


_DN = (((1,), (1,)), ((), ()))


def _make_kernel(tn):
    def _noisy_kernel(x_ref, wmu_ref, wsig_ref, eout_ref, ein_ref,
                      bmu_ref, bsig_ref, beps_ref, o_ref):
        j = pl.program_id(0)
        sl = pl.ds(j * tn, tn)
        eo = eout_ref[0, sl][:, None]
        eps = eo * ein_ref[...]
        w = (wmu_ref[...] + wsig_ref[...] * eps).astype(jnp.bfloat16)
        xb = x_ref[...].astype(jnp.bfloat16)
        acc = lax.dot_general(xb, w, _DN, preferred_element_type=jnp.float32)
        bias = bmu_ref[0, sl] + bsig_ref[0, sl] * beps_ref[0, sl]
        o_ref[...] = acc + bias[None, :]
    return _noisy_kernel


@jax.jit
def kernel(x, weight_mu, weight_sigma, eps_in, eps_out,
           bias_mu, bias_sigma, bias_epsilon):
    x = jnp.asarray(x, jnp.float32)
    B, I = x.shape
    O = bias_mu.shape[0]

    tn = min(_round_up(O, 256), 512)
    M, N, K = _round_up(B, 8), _round_up(O, tn), _round_up(I, 128)

    x_p = _maybe_pad2d(x, M, K)
    wmu = _maybe_pad2d(weight_mu, N, K)
    wsig = _maybe_pad2d(weight_sigma, N, K)
    eout = _maybe_pad2d(eps_out.reshape(1, O), 1, N)
    ein = _maybe_pad2d(eps_in.reshape(1, I), 1, K)
    bmu = _maybe_pad2d(bias_mu.reshape(1, O), 1, N)
    bsig = _maybe_pad2d(bias_sigma.reshape(1, O), 1, N)
    beps = _maybe_pad2d(bias_epsilon.reshape(1, O), 1, N)

    row_n = pl.BlockSpec((1, N), lambda j: (0, 0))
    grid = (N // tn,)
    out = pl.pallas_call(
        _make_kernel(tn),
        out_shape=jax.ShapeDtypeStruct((M, N), jnp.float32),
        grid=grid,
        in_specs=[
            pl.BlockSpec((M, K), lambda j: (0, 0)),
            pl.BlockSpec((tn, K), lambda j: (j, 0)),
            pl.BlockSpec((tn, K), lambda j: (j, 0)),
            row_n,
            pl.BlockSpec((1, K), lambda j: (0, 0)),
            row_n, row_n, row_n,
        ],
        out_specs=pl.BlockSpec((M, tn), lambda j: (0, j)),
        compiler_params=pltpu.CompilerParams(
            dimension_semantics=("parallel",)),
    )(x_p, wmu, wsig, eout, ein, bmu, bsig, beps)

    return out[:B, :O]
